# Initial kernel scaffold; baseline (speedup 1.0000x reference)
import jax
import jax.numpy as jnp
from jax.experimental import pallas as pl


def kernel(patch_w, patch_b, pool_w, pool_b, bott_w_main, bott_w_pool, bott_b, cls_w, cls_b, x):
    raise NotImplementedError("write your pallas kernel here")


# speedup vs baseline: 1.5435x; 1.5435x over previous
import functools

import numpy as np
import jax
import jax.numpy as jnp
from jax.experimental import pallas as pl
from jax.experimental.pallas import tpu as pltpu


def _interp_matrix_np(out_size, in_size):
    if in_size == 1:
        return np.ones((out_size, 1), np.float32)
    if out_size == 1:
        src = np.zeros((1,), np.float64)
    else:
        src = np.arange(out_size, dtype=np.float64) * (in_size - 1) / (out_size - 1)
    i0 = np.clip(np.floor(src).astype(np.int64), 0, in_size - 1)
    i1 = np.minimum(i0 + 1, in_size - 1)
    frac = (src - i0).astype(np.float32)
    R = np.zeros((out_size, in_size), np.float32)
    R[np.arange(out_size), i0] += 1.0 - frac
    R[np.arange(out_size), i1] += frac
    return R


def _mega_kernel(x_ref, ssel_ref, wsel_ref, pbt_ref, poolwt_ref, poolbt_ref,
                 wmt_ref, bwpt_ref, bbt_ref, cwt_ref, cbt_ref, rh_ref, rwt_ref,
                 o_ref, ft_ref, fs_ref, feats_ref, v3_ref, *, dims, inv_p):
    C, h, w, W = dims
    k = pl.program_id(1)

    @pl.when(k == 0)
    def _head():
        E = wsel_ref.shape[0]
        K = cwt_ref.shape[0]
        pb = jnp.broadcast_to(pbt_ref[...], (E, 2 * w))
        fs_ref[...] = jnp.zeros_like(fs_ref)

        def band(i2, _):
            a = x_ref[0, :, pl.ds(16 * i2, 16), :].astype(jnp.bfloat16)
            a = a.reshape(C * 16, W)
            a0 = jnp.concatenate([a[16 * c:16 * c + 8] for c in range(C)],
                                 axis=0)
            a1 = jnp.concatenate([a[16 * c + 8:16 * c + 16] for c in range(C)],
                                 axis=0)
            sel = jnp.concatenate(
                [jnp.concatenate(
                    [jnp.dot(ai, ssel_ref[dx],
                             preferred_element_type=jnp.float32)
                     .astype(jnp.bfloat16)
                     for dx in range(8)], axis=0)
                 for ai in (a0, a1)], axis=1)
            acc = jnp.dot(wsel_ref[...], sel,
                          preferred_element_type=jnp.float32)
            fband = jnp.maximum(acc + pb, 0.0)
            fs_ref[...] = fs_ref[...] + fband
            ft_ref[:, pl.ds(2 * w * i2, 2 * w)] = fband.astype(jnp.bfloat16)
            return 0

        jax.lax.fori_loop(0, h // 2, band, 0)

        mean = jnp.sum(fs_ref[...], axis=1, keepdims=True) * inv_p
        hdn = jnp.dot(poolwt_ref[...], mean,
                      preferred_element_type=jnp.float32) + poolbt_ref[...]
        hdn = jnp.maximum(hdn, 0.0)
        pc = jnp.dot(bwpt_ref[...], hdn,
                     preferred_element_type=jnp.float32) + bbt_ref[...]

        feats = jnp.dot(wmt_ref[...], ft_ref[...],
                        preferred_element_type=jnp.float32)
        feats_ref[...] = jnp.maximum(feats + pc, 0.0).astype(jnp.bfloat16)

        cb = jnp.broadcast_to(cbt_ref[...], (K, 2 * w))

        def cls_band(i2, _):
            lg = jnp.dot(cwt_ref[...], feats_ref[:, pl.ds(2 * w * i2, 2 * w)],
                         preferred_element_type=jnp.float32) + cb
            v0 = jnp.dot(lg[:, :w].astype(jnp.bfloat16), rwt_ref[...],
                         preferred_element_type=jnp.float32)
            v1 = jnp.dot(lg[:, w:].astype(jnp.bfloat16), rwt_ref[...],
                         preferred_element_type=jnp.float32)
            v3_ref[pl.ds(2 * i2, 1)] = v0.reshape(1, K, W)
            v3_ref[pl.ds(2 * i2 + 1, 1)] = v1.reshape(1, K, W)
            return 0

        jax.lax.fori_loop(0, h // 2, cls_band, 0)

    vk = v3_ref[:, pl.ds(k, 1), :].reshape(h, W)
    o_ref[0, 0] = jnp.dot(rh_ref[...], vk, preferred_element_type=jnp.float32)


def kernel(patch_w, patch_b, pool_w, pool_b, bott_w_main, bott_w_pool,
           bott_b, cls_w, cls_b, x):
    N, C, H, W = x.shape
    h, w = H // 8, W // 8
    P = h * w
    C64 = C * 64
    E = patch_w.shape[1]
    Pd = pool_w.shape[1]
    Bn = bott_w_main.shape[1]
    K = cls_w.shape[1]

    ssel_np = np.zeros((8, W, w), np.float32)
    for dx in range(8):
        ssel_np[dx, 8 * np.arange(w) + dx, np.arange(w)] = 1.0
    ssel = jnp.asarray(ssel_np, jnp.bfloat16)

    wsel = (patch_w.reshape(C, 8, 8, E).transpose(2, 0, 1, 3)
            .reshape(C64, E).T)

    pbt = patch_b.T
    poolwt = pool_w.T
    poolbt = pool_b.T
    wmt = bott_w_main.T
    bwpt = bott_w_pool.T
    bbt = bott_b.T
    cwt = cls_w.T
    cbt = cls_b.T

    Rh = jnp.asarray(_interp_matrix_np(H, h))
    Rw_T = jnp.asarray(_interp_matrix_np(W, w).T,
                       jnp.bfloat16)

    return pl.pallas_call(
        functools.partial(_mega_kernel, dims=(C, h, w, W),
                          inv_p=1.0 / float(P)),
        grid=(N, K),
        in_specs=[
            pl.BlockSpec((1, C, H, W), lambda n, k: (n, 0, 0, 0)),
            pl.BlockSpec((8, W, w), lambda n, k: (0, 0, 0)),
            pl.BlockSpec((E, C64), lambda n, k: (0, 0)),
            pl.BlockSpec((E, 1), lambda n, k: (0, 0)),
            pl.BlockSpec((Pd, E), lambda n, k: (0, 0)),
            pl.BlockSpec((Pd, 1), lambda n, k: (0, 0)),
            pl.BlockSpec((Bn, E), lambda n, k: (0, 0)),
            pl.BlockSpec((Bn, Pd), lambda n, k: (0, 0)),
            pl.BlockSpec((Bn, 1), lambda n, k: (0, 0)),
            pl.BlockSpec((K, Bn), lambda n, k: (0, 0)),
            pl.BlockSpec((K, 1), lambda n, k: (0, 0)),
            pl.BlockSpec((H, h), lambda n, k: (0, 0)),
            pl.BlockSpec((w, W), lambda n, k: (0, 0)),
        ],
        out_specs=pl.BlockSpec((1, 1, H, W), lambda n, k: (n, k, 0, 0)),
        out_shape=jax.ShapeDtypeStruct((N, K, H, W), jnp.float32),
        scratch_shapes=[
            pltpu.VMEM((E, P), jnp.bfloat16),
            pltpu.VMEM((E, 2 * w), jnp.float32),
            pltpu.VMEM((Bn, P), jnp.bfloat16),
            pltpu.VMEM((h, K, W), jnp.float32),
        ],
        compiler_params=pltpu.CompilerParams(
            dimension_semantics=("parallel", "arbitrary")),
    )(x, ssel, wsel, pbt, poolwt, poolbt, wmt, bwpt, bbt, cwt, cbt, Rh, Rw_T)


# speedup vs baseline: 2.6581x; 1.7221x over previous
import functools

import numpy as np
import jax
import jax.numpy as jnp
from jax.experimental import pallas as pl
from jax.experimental.pallas import tpu as pltpu


def _interp_matrix_np(out_size, in_size):
    if in_size == 1:
        return np.ones((out_size, 1), np.float32)
    if out_size == 1:
        src = np.zeros((1,), np.float64)
    else:
        src = np.arange(out_size, dtype=np.float64) * (in_size - 1) / (out_size - 1)
    i0 = np.clip(np.floor(src).astype(np.int64), 0, in_size - 1)
    i1 = np.minimum(i0 + 1, in_size - 1)
    frac = (src - i0).astype(np.float32)
    R = np.zeros((out_size, in_size), np.float32)
    R[np.arange(out_size), i0] += 1.0 - frac
    R[np.arange(out_size), i1] += frac
    return R


def _mega_kernel(x_ref, sall_ref, wselb_ref, poolwt_ref, poolbt_ref,
                 wmt_ref, bwpt_ref, bbt_ref, cwt_ref, cbt_ref, rh_ref, rwt_ref,
                 o_ref, selall_ref, selcat_ref, ft_ref, feats_ref, lg_ref,
                 v3_ref, *, dims, inv_p):
    C, h, w, H, W = dims
    C64 = C * 64
    k = pl.program_id(1)

    @pl.when(k == 0)
    def _head():
        E = ft_ref.shape[0]
        K = cwt_ref.shape[0]
        P = h * w

        x2 = x_ref[0].reshape(C * H, W).astype(jnp.bfloat16)
        selall_ref[...] = jnp.dot(x2, sall_ref[...],
                                  preferred_element_type=jnp.float32
                                  ).astype(jnp.bfloat16)

        selcat_ref[pl.ds(C64, 8), :] = jnp.ones((8, P), jnp.bfloat16)

        def band(i2, _):
            blks = [selall_ref[pl.ds(c * H + 16 * i2, 16), :] for c in range(C)]
            rows = []
            for dx in range(8):
                for c in range(C):
                    b = blks[c]
                    rows.append(jnp.concatenate(
                        [b[:8, w * dx:w * (dx + 1)],
                         b[8:, w * dx:w * (dx + 1)]], axis=1))
            selcat_ref[:C64, pl.ds(128 * i2, 128)] = jnp.concatenate(rows, axis=0)
            return 0

        jax.lax.fori_loop(0, h // 2, band, 0)

        f = jnp.maximum(jnp.dot(wselb_ref[...], selcat_ref[...],
                                preferred_element_type=jnp.float32), 0.0)
        ft_ref[...] = f.astype(jnp.bfloat16)

        mean = jnp.sum(f, axis=1, keepdims=True) * inv_p
        hdn = jnp.dot(poolwt_ref[...], mean,
                      preferred_element_type=jnp.float32) + poolbt_ref[...]
        hdn = jnp.maximum(hdn, 0.0)
        pc = jnp.dot(bwpt_ref[...], hdn,
                     preferred_element_type=jnp.float32) + bbt_ref[...]

        feats = jnp.dot(wmt_ref[...], ft_ref[...],
                        preferred_element_type=jnp.float32)
        feats_ref[...] = jnp.maximum(feats + pc, 0.0).astype(jnp.bfloat16)
        lg_ref[...] = jnp.dot(cwt_ref[...], feats_ref[...],
                              preferred_element_type=jnp.float32) + cbt_ref[...]

        def colpass(i2, _):
            lgp = lg_ref[:, pl.ds(128 * i2, 128)]
            v0 = jnp.dot(lgp[:, :w].astype(jnp.bfloat16), rwt_ref[...],
                         preferred_element_type=jnp.float32)
            v1 = jnp.dot(lgp[:, w:].astype(jnp.bfloat16), rwt_ref[...],
                         preferred_element_type=jnp.float32)
            v3_ref[pl.ds(2 * i2, 1)] = v0.reshape(1, K, W)
            v3_ref[pl.ds(2 * i2 + 1, 1)] = v1.reshape(1, K, W)
            return 0

        jax.lax.fori_loop(0, h // 2, colpass, 0)

    vk = v3_ref[:, pl.ds(k, 1), :].reshape(h, W)
    o_ref[0, 0] = jnp.dot(rh_ref[...], vk, preferred_element_type=jnp.float32)


def kernel(patch_w, patch_b, pool_w, pool_b, bott_w_main, bott_w_pool,
           bott_b, cls_w, cls_b, x):
    N, C, H, W = x.shape
    h, w = H // 8, W // 8
    P = h * w
    C64 = C * 64
    E = patch_w.shape[1]
    Pd = pool_w.shape[1]
    Bn = bott_w_main.shape[1]
    K = cls_w.shape[1]

    sall_np = np.zeros((W, W), np.float32)
    for dx in range(8):
        sall_np[8 * np.arange(w) + dx, dx * w + np.arange(w)] = 1.0
    sall = jnp.asarray(sall_np, jnp.bfloat16)

    wsel = (patch_w.reshape(C, 8, 8, E).transpose(2, 0, 1, 3)
            .reshape(C64, E).T)
    wselb = jnp.concatenate(
        [wsel, patch_b.T.astype(jnp.bfloat16),
         jnp.zeros((E, 7), jnp.bfloat16)], axis=1)

    poolwt = pool_w.T
    poolbt = pool_b.T
    wmt = bott_w_main.T
    bwpt = bott_w_pool.T
    bbt = bott_b.T
    cwt = cls_w.T
    cbt = cls_b.T

    Rh = jnp.asarray(_interp_matrix_np(H, h))
    Rw_T = jnp.asarray(_interp_matrix_np(W, w).T,
                       jnp.bfloat16)

    return pl.pallas_call(
        functools.partial(_mega_kernel, dims=(C, h, w, H, W),
                          inv_p=1.0 / float(P)),
        grid=(N, K),
        in_specs=[
            pl.BlockSpec((1, C, H, W), lambda n, k: (n, 0, 0, 0)),
            pl.BlockSpec((W, W), lambda n, k: (0, 0)),
            pl.BlockSpec((E, C64 + 8), lambda n, k: (0, 0)),
            pl.BlockSpec((Pd, E), lambda n, k: (0, 0)),
            pl.BlockSpec((Pd, 1), lambda n, k: (0, 0)),
            pl.BlockSpec((Bn, E), lambda n, k: (0, 0)),
            pl.BlockSpec((Bn, Pd), lambda n, k: (0, 0)),
            pl.BlockSpec((Bn, 1), lambda n, k: (0, 0)),
            pl.BlockSpec((K, Bn), lambda n, k: (0, 0)),
            pl.BlockSpec((K, 1), lambda n, k: (0, 0)),
            pl.BlockSpec((H, h), lambda n, k: (0, 0)),
            pl.BlockSpec((w, W), lambda n, k: (0, 0)),
        ],
        out_specs=pl.BlockSpec((1, 1, H, W), lambda n, k: (n, k, 0, 0)),
        out_shape=jax.ShapeDtypeStruct((N, K, H, W), jnp.float32),
        scratch_shapes=[
            pltpu.VMEM((C * H, W), jnp.bfloat16),
            pltpu.VMEM((C64 + 8, P), jnp.bfloat16),
            pltpu.VMEM((E, P), jnp.bfloat16),
            pltpu.VMEM((Bn, P), jnp.bfloat16),
            pltpu.VMEM((K, P), jnp.float32),
            pltpu.VMEM((h, K, W), jnp.float32),
        ],
        compiler_params=pltpu.CompilerParams(
            dimension_semantics=("parallel", "arbitrary")),
    )(x, sall, wselb, poolwt, poolbt, wmt, bwpt, bbt, cwt, cbt, Rh, Rw_T)
